# initial kernel scaffold (unmeasured)
import jax
import jax.numpy as jnp
from jax import lax
from jax.experimental import pallas as pl
from jax.experimental.pallas import tpu as pltpu

N_DEV = 4
BM = 1024
EPS = 1e-5


def kernel(x, gamma):
    M, Nsh = x.shape
    NBLK = M // BM
    global_n = Nsh * N_DEV
    gamma2 = gamma.reshape(1, Nsh)

    def body(x_ref, g_ref, out_ref, cache, pfull, comm, obuf,
             send_sems, recv_sems, out_sems):
        b = pl.program_id(0)
        my = lax.axis_index("i")

        xb = x_ref[...]
        xsq = xb * xb
        ones_row = jnp.ones((1, Nsh), dtype=jnp.float32)
        s_row = lax.dot_general(
            ones_row, xsq,
            dimension_numbers=(((1,), (1,)), ((), ())),
            preferred_element_type=jnp.float32,
        )
        pfull[pl.ds(b, 1), :] = s_row
        cache[pl.ds(b * BM, BM), :] = xb.astype(jnp.bfloat16)

        @pl.when(b == NBLK - 1)
        def _():
            bsem = pltpu.get_barrier_semaphore()
            for k in range(1, N_DEV):
                pl.semaphore_signal(
                    bsem, inc=1,
                    device_id=((my + k) % N_DEV,),
                    device_id_type=pl.DeviceIdType.MESH,
                )
            pl.semaphore_wait(bsem, N_DEV - 1)

            rdmas = []
            for k in range(1, N_DEV):
                q = k - 1
                rd = pltpu.make_async_remote_copy(
                    src_ref=pfull,
                    dst_ref=comm.at[q],
                    send_sem=send_sems.at[q],
                    recv_sem=recv_sems.at[q],
                    device_id=((my + k) % N_DEV,),
                    device_id_type=pl.DeviceIdType.MESH,
                )
                rd.start()
                rdmas.append(rd)
            for rd in rdmas:
                rd.wait()

            total = pfull[...] + comm[0] + comm[1] + comm[2]
            inv = lax.rsqrt(total * (1.0 / global_n) + EPS)
            inv_t = inv.T
            g = g_ref[...]

            copies = [None, None]
            for i in range(NBLK):
                slot = i % 2
                if copies[slot] is not None:
                    copies[slot].wait()
                cb = cache[pl.ds(i * BM, BM), :].astype(jnp.float32)
                inv_col = inv_t[:, i:i + 1]
                ob = cb * inv_col * g
                obuf[slot] = ob.astype(jnp.bfloat16)
                cp = pltpu.make_async_copy(
                    obuf.at[slot],
                    out_ref.at[pl.ds(i * BM, BM), :],
                    out_sems.at[slot],
                )
                cp.start()
                copies[slot] = cp
            for cp in copies:
                if cp is not None:
                    cp.wait()

    return pl.pallas_call(
        body,
        grid=(NBLK,),
        in_specs=[
            pl.BlockSpec((BM, Nsh), lambda b: (b, 0)),
            pl.BlockSpec((1, Nsh), lambda b: (0, 0)),
        ],
        out_specs=pl.BlockSpec(memory_space=pltpu.ANY),
        out_shape=jax.ShapeDtypeStruct((M, Nsh), jnp.bfloat16),
        scratch_shapes=[
            pltpu.VMEM((M, Nsh), jnp.bfloat16),
            pltpu.VMEM((8, BM), jnp.float32),
            pltpu.VMEM((3, 8, BM), jnp.float32),
            pltpu.VMEM((2, BM, Nsh), jnp.bfloat16),
            pltpu.SemaphoreType.DMA((3,)),
            pltpu.SemaphoreType.DMA((3,)),
            pltpu.SemaphoreType.DMA((2,)),
        ],
        compiler_params=pltpu.CompilerParams(collective_id=0),
    )(x, gamma2)


# baseline (device time: 63244 ns/iter reference)
import jax
import jax.numpy as jnp
from jax import lax
from jax.experimental import pallas as pl
from jax.experimental.pallas import tpu as pltpu

N_DEV = 4
BM = 1024
EPS = 1e-5


def kernel(x, gamma):
    M, Nsh = x.shape
    NBLK = M // BM
    global_n = Nsh * N_DEV
    gamma2 = gamma.reshape(1, Nsh)

    def body(x_ref, g_ref, out_ref, cache, pfull, comm, obuf,
             send_sems, recv_sems, out_sems):
        b = pl.program_id(0)
        my = lax.axis_index("i")

        xb = x_ref[...]
        xsq = xb * xb
        ones_row = jnp.ones((1, Nsh), dtype=jnp.float32)
        s_row = lax.dot_general(
            ones_row, xsq,
            dimension_numbers=(((1,), (1,)), ((), ())),
            preferred_element_type=jnp.float32,
        )
        pfull[pl.ds(b, 1), :] = s_row
        cache[pl.ds(b * BM, BM), :] = xb.astype(jnp.bfloat16)

        @pl.when(b == NBLK - 1)
        def _():
            bsem = pltpu.get_barrier_semaphore()
            for k in range(1, N_DEV):
                pl.semaphore_signal(
                    bsem, inc=1,
                    device_id=((my + k) % N_DEV,),
                    device_id_type=pl.DeviceIdType.MESH,
                )
            pl.semaphore_wait(bsem, N_DEV - 1)

            rdmas = []
            for k in range(1, N_DEV):
                q = k - 1
                rd = pltpu.make_async_remote_copy(
                    src_ref=pfull,
                    dst_ref=comm.at[q],
                    send_sem=send_sems.at[q],
                    recv_sem=recv_sems.at[q],
                    device_id=((my + k) % N_DEV,),
                    device_id_type=pl.DeviceIdType.MESH,
                )
                rd.start()
                rdmas.append(rd)
            for rd in rdmas:
                rd.wait()

            total = pfull[...] + comm[0] + comm[1] + comm[2]
            inv = lax.rsqrt(total * (1.0 / global_n) + EPS)
            inv_t = inv.T
            g = g_ref[...]

            copies = [None, None]
            for i in range(NBLK):
                slot = i % 2
                if copies[slot] is not None:
                    copies[slot].wait()
                cb = cache[pl.ds(i * BM, BM), :].astype(jnp.float32)
                inv_col = inv_t[:, i:i + 1]
                ob = cb * inv_col * g
                obuf[slot] = ob.astype(jnp.bfloat16)
                cp = pltpu.make_async_copy(
                    obuf.at[slot],
                    out_ref.at[pl.ds(i * BM, BM), :],
                    out_sems.at[slot],
                )
                cp.start()
                copies[slot] = cp
            for cp in copies:
                if cp is not None:
                    cp.wait()

    return pl.pallas_call(
        body,
        grid=(NBLK,),
        in_specs=[
            pl.BlockSpec((BM, Nsh), lambda b: (b, 0)),
            pl.BlockSpec((1, Nsh), lambda b: (0, 0)),
        ],
        out_specs=pl.BlockSpec(memory_space=pl.ANY),
        out_shape=jax.ShapeDtypeStruct((M, Nsh), jnp.bfloat16),
        scratch_shapes=[
            pltpu.VMEM((M, Nsh), jnp.bfloat16),
            pltpu.VMEM((8, BM), jnp.float32),
            pltpu.VMEM((3, 8, BM), jnp.float32),
            pltpu.VMEM((2, BM, Nsh), jnp.bfloat16),
            pltpu.SemaphoreType.DMA((3,)),
            pltpu.SemaphoreType.DMA((3,)),
            pltpu.SemaphoreType.DMA((2,)),
        ],
        compiler_params=pltpu.CompilerParams(
            collective_id=0,
            vmem_limit_bytes=63 * 1024 * 1024,
        ),
    )(x, gamma2)


# device time: 59260 ns/iter; 1.0672x vs baseline; 1.0672x over previous
import jax
import jax.numpy as jnp
from jax import lax
from jax.experimental import pallas as pl
from jax.experimental.pallas import tpu as pltpu

N_DEV = 4
BM = 1024
EPS = 1e-5


def kernel(x, gamma):
    M, Nsh = x.shape
    NBLK = M // BM
    global_n = Nsh * N_DEV
    gamma2 = gamma.reshape(1, Nsh)

    def body(x_ref, g_ref, out_ref, cache, pfull, comm, obuf,
             send_sems, recv_sems, out_sems):
        b = pl.program_id(0)
        my = lax.axis_index("i")
        bsem = pltpu.get_barrier_semaphore()

        @pl.when(b == 0)
        def _():
            for k in range(1, N_DEV):
                pl.semaphore_signal(
                    bsem, inc=1,
                    device_id=((my + k) % N_DEV,),
                    device_id_type=pl.DeviceIdType.MESH,
                )

        xb = x_ref[...]
        y = xb.reshape(8, 128, Nsh)
        s = jnp.sum(y * y, axis=2)
        pfull[pl.ds(8 * b, 8), :] = s
        cache[pl.ds(b * BM, BM), :] = xb.astype(jnp.bfloat16)

        @pl.when(b == NBLK - 1)
        def _():
            pl.semaphore_wait(bsem, N_DEV - 1)

            rdmas = []
            for k in range(1, N_DEV):
                q = k - 1
                rd = pltpu.make_async_remote_copy(
                    src_ref=pfull,
                    dst_ref=comm.at[q],
                    send_sem=send_sems.at[q],
                    recv_sem=recv_sems.at[q],
                    device_id=((my + k) % N_DEV,),
                    device_id_type=pl.DeviceIdType.MESH,
                )
                rd.start()
                rdmas.append(rd)
            for rd in rdmas:
                rd.wait()

            total = pfull[...] + comm[0] + comm[1] + comm[2]
            inv = lax.rsqrt(total * (1.0 / global_n) + EPS)
            invb = inv.astype(jnp.bfloat16)
            g3 = g_ref[...].astype(jnp.bfloat16).reshape(1, 1, Nsh)

            copies = [None, None]
            for i in range(NBLK):
                slot = i % 2
                if copies[slot] is not None:
                    copies[slot].wait()
                cb = cache[pl.ds(i * BM, BM), :]
                y3 = cb.reshape(8, 128, Nsh)
                inv_b = invb[8 * i:8 * (i + 1), :]
                ob = y3 * inv_b[:, :, None] * g3
                obuf[slot] = ob.reshape(BM, Nsh)
                cp = pltpu.make_async_copy(
                    obuf.at[slot],
                    out_ref.at[pl.ds(i * BM, BM), :],
                    out_sems.at[slot],
                )
                cp.start()
                copies[slot] = cp
            for cp in copies:
                if cp is not None:
                    cp.wait()

    return pl.pallas_call(
        body,
        grid=(NBLK,),
        in_specs=[
            pl.BlockSpec((BM, Nsh), lambda b: (b, 0)),
            pl.BlockSpec((1, Nsh), lambda b: (0, 0)),
        ],
        out_specs=pl.BlockSpec(memory_space=pl.ANY),
        out_shape=jax.ShapeDtypeStruct((M, Nsh), jnp.bfloat16),
        scratch_shapes=[
            pltpu.VMEM((M, Nsh), jnp.bfloat16),
            pltpu.VMEM((64, 128), jnp.float32),
            pltpu.VMEM((3, 64, 128), jnp.float32),
            pltpu.VMEM((2, BM, Nsh), jnp.bfloat16),
            pltpu.SemaphoreType.DMA((3,)),
            pltpu.SemaphoreType.DMA((3,)),
            pltpu.SemaphoreType.DMA((2,)),
        ],
        compiler_params=pltpu.CompilerParams(
            collective_id=0,
            vmem_limit_bytes=63 * 1024 * 1024,
        ),
    )(x, gamma2)
